# baseline (speedup 1.0000x reference)
import jax
import jax.numpy as jnp
from jax.experimental import pallas as pl


_N = 4 * 1024 * 1024
_C = 4
_R = _N // 256
_L = 256 * _C
_BLK = 1024


def _one_hot_body(o_ref):
    pid = pl.program_id(0)
    blocks_per_layer = (_R // _BLK) // _C
    lid = pid // blocks_per_layer
    lane = jax.lax.broadcasted_iota(jnp.int32, (_BLK, _L), 1)
    o_ref[...] = ((lane & (_C - 1)) == lid).astype(jnp.int32)


def kernel(w0, w1, w2, w3, y):
    out = pl.pallas_call(
        _one_hot_body,
        grid=(_R // _BLK,),
        out_specs=pl.BlockSpec((_BLK, _L), lambda i: (i, 0)),
        out_shape=jax.ShapeDtypeStruct((_R, _L), jnp.int32),
    )()
    one_hot = out.reshape(_N, _C).astype(jnp.int64)
    return (one_hot, y)


# speedup vs baseline: 1.7217x; 1.7217x over previous
import jax
import jax.numpy as jnp
from jax.experimental import pallas as pl


_N = 4 * 1024 * 1024
_C = 4
_BLK = 32768


def _one_hot_body(o_ref):
    pid = pl.program_id(0)
    blocks_per_layer = (_N // _BLK) // _C
    lid = pid // blocks_per_layer
    lane = jax.lax.broadcasted_iota(jnp.int32, (_BLK, _C), 1)
    o_ref[...] = (lane == lid).astype(jnp.int32)


def kernel(w0, w1, w2, w3, y):
    out = pl.pallas_call(
        _one_hot_body,
        grid=(_N // _BLK,),
        out_specs=pl.BlockSpec((_BLK, _C), lambda i: (i, 0)),
        out_shape=jax.ShapeDtypeStruct((_N, _C), jnp.int32),
    )()
    return (out.astype(jnp.int64), y)
